# baseline (speedup 1.0000x reference)
import functools

import jax
import jax.numpy as jnp
from jax import lax
from jax.experimental import pallas as pl
from jax.experimental.pallas import tpu as pltpu
from jax.experimental.pallas import tpu_sc as plsc

N_TOTAL = 4096 * 4096
NCOL = 4096
NC = 2
NS = 16
L = 16
NW = NC * NS
ROWS_PER_TILE = 4096 // NW
CHUNK_ROWS = 8
NCHUNK = ROWS_PER_TILE // CHUNK_ROWS
VPR = NCOL // L
REP = 8
TBL = 2 * REP * 4 * L
MAGIC = 201326592.0


def _body(w_hbm, par_hbm, out_hbm, buf0, buf1, par, cnt_t, sum_t, ssq_t,
          dif_t, sem0, sem1):
    wid = lax.axis_index("s") * NC + lax.axis_index("c")
    base = wid * ROWS_PER_TILE

    pltpu.sync_copy(par_hbm, par)

    zeros = jnp.zeros((L,), jnp.float32)
    for k in range(TBL // L):
        cnt_t[pl.ds(k * L, L)] = zeros
        sum_t[pl.ds(k * L, L)] = zeros
        ssq_t[pl.ds(k * L, L)] = zeros
        dif_t[pl.ds(k * L, L)] = zeros

    ia16 = par[pl.ds(0, L)]
    thr = par[pl.ds(2 * L, L)]
    lane = par[pl.ds(3 * L, L)]
    lanes = [lane + jnp.float32(64.0 * j) for j in range(REP)]
    ones = jnp.full((L,), 1.0, jnp.float32)
    magic = jnp.full((L,), MAGIC, jnp.float32)
    hi = jnp.full((L,), 16.0, jnp.float32)
    lo = jnp.full((L,), -32.0, jnp.float32)
    far = jnp.full((L,), 512.0, jnp.float32)
    fzero = jnp.zeros((L,), jnp.float32)

    def start_dma(g, b):
        return pltpu.async_copy(
            w_hbm.at[pl.ds(base + g * CHUNK_ROWS, CHUNK_ROWS), :],
            buf1 if b else buf0, sem1 if b else sem0)

    def wait_dma(g, b):
        pltpu.make_async_copy(
            w_hbm.at[pl.ds(base + g * CHUNK_ROWS, CHUNK_ROWS), :],
            buf1 if b else buf0, sem1 if b else sem0).wait()

    def compute_chunk(bufb):
        @plsc.parallel_loop(0, VPR, step=1, unroll=1)
        def chunk_body(i):
            for j in range(REP):
                w = bufb[j, pl.ds(i * L, L)]
                tu = w * ia16
                t = jnp.minimum(tu, hi)
                t = jnp.maximum(t, lo)
                b16 = (t + magic) - magic
                d16 = jnp.abs(tu - b16)
                w2 = w * w
                nsel = jnp.where(d16 < thr, fzero, far)
                ai = ((b16 + lanes[j]) + nsel).astype(jnp.int32)
                plsc.addupdate_scatter(cnt_t, [ai], ones)
                plsc.addupdate_scatter(sum_t, [ai], w)
                plsc.addupdate_scatter(ssq_t, [ai], w2)
                plsc.addupdate_scatter(dif_t, [ai], d16)

    start_dma(0, 0)
    start_dma(1, 1)

    def outer(t, carry):
        g0 = 2 * t
        wait_dma(g0, 0)
        compute_chunk(buf0)

        @pl.when(g0 + 2 < NCHUNK)
        def _():
            start_dma(g0 + 2, 0)

        wait_dma(g0 + 1, 1)
        compute_chunk(buf1)

        @pl.when(g0 + 3 < NCHUNK)
        def _():
            start_dma(g0 + 3, 1)

        return carry

    lax.fori_loop(0, NCHUNK // 2, outer, 0)

    pltpu.sync_copy(cnt_t, out_hbm.at[wid, pl.ds(0, TBL)])
    pltpu.sync_copy(sum_t, out_hbm.at[wid, pl.ds(TBL, TBL)])
    pltpu.sync_copy(ssq_t, out_hbm.at[wid, pl.ds(2 * TBL, TBL)])
    pltpu.sync_copy(dif_t, out_hbm.at[wid, pl.ds(3 * TBL, TBL)])


@jax.jit
def _run(w, params):
    mesh = plsc.VectorSubcoreMesh(core_axis_name="c", subcore_axis_name="s")
    return pl.kernel(
        _body,
        out_type=jax.ShapeDtypeStruct((NW, 4 * TBL), jnp.float32),
        mesh=mesh,
        compiler_params=pltpu.CompilerParams(
            needs_layout_passes=False, use_tc_tiling_on_sc=True),
        scratch_types=[
            pltpu.VMEM((CHUNK_ROWS, NCOL), jnp.float32),
            pltpu.VMEM((CHUNK_ROWS, NCOL), jnp.float32),
            pltpu.VMEM((4 * L,), jnp.float32),
            pltpu.VMEM((TBL,), jnp.float32),
            pltpu.VMEM((TBL,), jnp.float32),
            pltpu.VMEM((TBL,), jnp.float32),
            pltpu.VMEM((TBL,), jnp.float32),
            pltpu.SemaphoreType.DMA,
            pltpu.SemaphoreType.DMA,
        ],
    )(w, params)


def kernel(weights, alpha):
    a = alpha.reshape(())
    a_s = lax.stop_gradient(a)
    params = jnp.concatenate([
        jnp.full((L,), 16.0, jnp.float32) / a_s,
        jnp.full((L,), 1.0 / 16.0, jnp.float32) * a_s,
        jnp.full((L,), 16.0, jnp.float32) * (0.01 * a_s),
        jnp.arange(L, dtype=jnp.float32) + 32.0,
    ])
    part = _run(weights, params)

    cnt_r = part[:, 0:TBL].reshape(NW, 2, REP, 4, L)
    cnt = cnt_r.sum(axis=(0, 1, 2, 4))
    near = cnt_r[:, 0].sum()
    sums = part[:, TBL:2 * TBL].reshape(NW, 2, REP, 4, L).sum(axis=(0, 1, 2, 4))
    ssq = part[:, 2 * TBL:3 * TBL].reshape(NW, 2, REP, 4, L).sum(
        axis=(0, 1, 2, 4))
    sdiff = part[:, 3 * TBL:4 * TBL].sum() * (a_s / 16.0)

    levels = jnp.arange(-2, 2, dtype=jnp.float32) * a_s
    safe_counts = jnp.maximum(cnt, 1.0)
    means = sums / safe_counts
    mse_per_bin = jnp.where(cnt > 0, (means - levels) ** 2, 0.0)
    var_per_bin = jnp.where(cnt >= 2.0, ssq / safe_counts - means ** 2, 0.0)
    total_mse = jnp.sum(mse_per_bin)
    total_var = jnp.sum(var_per_bin)
    loss = total_mse + total_var

    n = jnp.float32(N_TOTAL)
    sumdiff2 = jnp.sum(ssq - 2.0 * levels * sums + levels * levels * cnt)
    quantization_mse = sumdiff2 / n
    mean_distance = sdiff / n
    max_dist = a_s * 0.5
    effectiveness = jnp.clip(
        100.0 * (1.0 - mean_distance / (max_dist + 1e-12)), 0.0, 100.0)
    near_levels = near / n * 100.0

    return (loss, total_mse, total_var, quantization_mse, mean_distance,
            effectiveness, near_levels)


# speedup vs baseline: 1.0507x; 1.0507x over previous
import functools

import jax
import jax.numpy as jnp
from jax import lax
from jax.experimental import pallas as pl
from jax.experimental.pallas import tpu as pltpu
from jax.experimental.pallas import tpu_sc as plsc

N_TOTAL = 4096 * 4096
NCOL = 4096
NC = 2
NS = 16
L = 16
NW = NC * NS
ROWS_PER_TILE = 4096 // NW
CHUNK_ROWS = 8
NCHUNK = ROWS_PER_TILE // CHUNK_ROWS
VPR = NCOL // L
REP = 8
TBL = REP * 4 * L
MAGIC = 201326592.0


def _body(w_hbm, par_hbm, out_hbm, buf0, buf1, par, cnt_t, sum_t, ssq_t, acc,
          sem0, sem1):
    wid = lax.axis_index("s") * NC + lax.axis_index("c")
    base = wid * ROWS_PER_TILE

    pltpu.sync_copy(par_hbm, par)

    zeros = jnp.zeros((L,), jnp.float32)
    for k in range(TBL // L):
        cnt_t[pl.ds(k * L, L)] = zeros
        sum_t[pl.ds(k * L, L)] = zeros
        ssq_t[pl.ds(k * L, L)] = zeros

    ia16 = par[pl.ds(0, L)]
    thr = par[pl.ds(2 * L, L)]
    lane = par[pl.ds(3 * L, L)]
    lanes = [lane + jnp.float32(64.0 * j) for j in range(REP)]
    ones = jnp.full((L,), 1.0, jnp.float32)
    magic = jnp.full((L,), MAGIC, jnp.float32)
    hi = jnp.full((L,), 16.0, jnp.float32)
    lo = jnp.full((L,), -32.0, jnp.float32)

    def start_dma(g, b):
        return pltpu.async_copy(
            w_hbm.at[pl.ds(base + g * CHUNK_ROWS, CHUNK_ROWS), :],
            buf1 if b else buf0, sem1 if b else sem0)

    def wait_dma(g, b):
        pltpu.make_async_copy(
            w_hbm.at[pl.ds(base + g * CHUNK_ROWS, CHUNK_ROWS), :],
            buf1 if b else buf0, sem1 if b else sem0).wait()

    def compute_chunk(bufb, lanes_b, carry):
        @plsc.parallel_loop(0, VPR, step=1, unroll=1, carry=carry)
        def chunk_body(i, c):
            sd = list(c[:4])
            nr = list(c[4:])
            for j in range(REP):
                w = bufb[j, pl.ds(i * L, L)]
                tu = w * ia16
                t = jnp.minimum(tu, hi)
                t = jnp.maximum(t, lo)
                b16 = (t + magic) - magic
                d16 = jnp.abs(tu - b16)
                w2 = w * w
                ai = (b16 + lanes_b[j]).astype(jnp.int32)
                plsc.addupdate_scatter(cnt_t, [ai], ones)
                plsc.addupdate_scatter(sum_t, [ai], w)
                plsc.addupdate_scatter(ssq_t, [ai], w2)
                sd[j % 4] = sd[j % 4] + d16
                nr[j % 4] = nr[j % 4] + jnp.where(d16 < thr, ones, 0.0)
            return (*sd, *nr)

        return chunk_body

    start_dma(0, 0)
    start_dma(1, 1)

    def outer(t, carry):
        g0 = 2 * t
        wait_dma(g0, 0)
        carry = compute_chunk(buf0, lanes, carry)

        @pl.when(g0 + 2 < NCHUNK)
        def _():
            start_dma(g0 + 2, 0)

        wait_dma(g0 + 1, 1)
        carry = compute_chunk(buf1, lanes, carry)

        @pl.when(g0 + 3 < NCHUNK)
        def _():
            start_dma(g0 + 3, 1)

        return carry

    fin = lax.fori_loop(
        0, NCHUNK // 2, outer,
        (zeros, zeros, zeros, zeros, zeros, zeros, zeros, zeros))
    sdiff = (fin[0] + fin[1]) + (fin[2] + fin[3])
    near = (fin[4] + fin[5]) + (fin[6] + fin[7])

    acc[pl.ds(0, L)] = sdiff
    acc[pl.ds(L, L)] = near

    pltpu.sync_copy(cnt_t, out_hbm.at[wid, pl.ds(0, TBL)])
    pltpu.sync_copy(sum_t, out_hbm.at[wid, pl.ds(TBL, TBL)])
    pltpu.sync_copy(ssq_t, out_hbm.at[wid, pl.ds(2 * TBL, TBL)])
    pltpu.sync_copy(acc, out_hbm.at[wid, pl.ds(3 * TBL, 2 * L)])


@jax.jit
def _run(w_flat, params):
    mesh = plsc.VectorSubcoreMesh(core_axis_name="c", subcore_axis_name="s")
    return pl.kernel(
        _body,
        out_type=jax.ShapeDtypeStruct((NW, 3 * TBL + 2 * L), jnp.float32),
        mesh=mesh,
        compiler_params=pltpu.CompilerParams(
            needs_layout_passes=False, use_tc_tiling_on_sc=True),
        scratch_types=[
            pltpu.VMEM((CHUNK_ROWS, NCOL), jnp.float32),
            pltpu.VMEM((CHUNK_ROWS, NCOL), jnp.float32),
            pltpu.VMEM((4 * L,), jnp.float32),
            pltpu.VMEM((TBL,), jnp.float32),
            pltpu.VMEM((TBL,), jnp.float32),
            pltpu.VMEM((TBL,), jnp.float32),
            pltpu.VMEM((2 * L,), jnp.float32),
            pltpu.SemaphoreType.DMA,
            pltpu.SemaphoreType.DMA,
        ],
    )(w_flat, params)


def kernel(weights, alpha):
    a = alpha.reshape(())
    a_s = lax.stop_gradient(a)
    params = jnp.concatenate([
        jnp.full((L,), 16.0, jnp.float32) / a_s,
        jnp.full((L,), 1.0 / 16.0, jnp.float32) * a_s,
        jnp.full((L,), 16.0, jnp.float32) * (0.01 * a_s),
        jnp.arange(L, dtype=jnp.float32) + 32.0,
    ])
    part = _run(weights, params)

    cnt = part[:, 0:TBL].reshape(NW, REP, 4, L).sum(axis=(0, 1, 3))
    sums = part[:, TBL:2 * TBL].reshape(NW, REP, 4, L).sum(axis=(0, 1, 3))
    ssq = part[:, 2 * TBL:3 * TBL].reshape(NW, REP, 4, L).sum(axis=(0, 1, 3))
    sdiff = part[:, 3 * TBL:3 * TBL + L].sum() * (a_s / 16.0)
    near = part[:, 3 * TBL + L:3 * TBL + 2 * L].sum()

    levels = jnp.arange(-2, 2, dtype=jnp.float32) * a_s
    safe_counts = jnp.maximum(cnt, 1.0)
    means = sums / safe_counts
    mse_per_bin = jnp.where(cnt > 0, (means - levels) ** 2, 0.0)
    var_per_bin = jnp.where(cnt >= 2.0, ssq / safe_counts - means ** 2, 0.0)
    total_mse = jnp.sum(mse_per_bin)
    total_var = jnp.sum(var_per_bin)
    loss = total_mse + total_var

    n = jnp.float32(N_TOTAL)
    sumdiff2 = jnp.sum(ssq - 2.0 * levels * sums + levels * levels * cnt)
    quantization_mse = sumdiff2 / n
    mean_distance = sdiff / n
    max_dist = a_s * 0.5
    effectiveness = jnp.clip(
        100.0 * (1.0 - mean_distance / (max_dist + 1e-12)), 0.0, 100.0)
    near_levels = near / n * 100.0

    return (loss, total_mse, total_var, quantization_mse, mean_distance,
            effectiveness, near_levels)
